# baseline (speedup 1.0000x reference)
import functools

import jax
import jax.numpy as jnp
from jax import lax
from jax.experimental import pallas as pl
from jax.experimental.pallas import tpu as pltpu
from jax.experimental.pallas import tpu_sc as plsc

K = 16
M_TILE = 4096
NSLICES = M_TILE // 128
Q_BLK = 512
DEC_BLK = 2048
NEG = -3.0e38
IMAX = 2147483647


def _topk_body(m_tiles, q_ref, w1_ref, b1_ref, w2_ref, b2_ref,
               g_ref, bb_ref, bank_ref, imp_ref, bias_ref, vals_ref, idx_ref,
               zn_ref, s_ref, bv_ref, bi_ref):
    m = pl.program_id(1)

    @pl.when(m == 0)
    def _():
        h = jax.nn.gelu(jnp.dot(q_ref[...], w1_ref[...],
                                preferred_element_type=jnp.float32) + b1_ref[...])
        z = jnp.dot(h, w2_ref[...], preferred_element_type=jnp.float32) + b2_ref[...]
        mu = jnp.mean(z, axis=-1, keepdims=True)
        var = jnp.mean((z - mu) ** 2, axis=-1, keepdims=True)
        z = (z - mu) / jnp.sqrt(var + 1e-5) * g_ref[...] + bb_ref[...]
        zn_ref[...] = z / (jnp.sqrt(jnp.sum(z * z, axis=-1, keepdims=True)) + 1e-8)
        bv_ref[...] = jnp.full((Q_BLK, K), NEG, jnp.float32)
        bi_ref[...] = jnp.zeros((Q_BLK, K), jnp.int32)

    bank = bank_ref[...]
    inv = 1.0 / (jnp.sqrt(jnp.sum(bank * bank, axis=-1, keepdims=True)) + 1e-8)
    mn = bank * inv
    s = lax.dot_general(zn_ref[...], mn, (((1,), (1,)), ((), ())),
                        preferred_element_type=jnp.float32)
    s_ref[...] = s * (0.5 + 0.5 * imp_ref[...]) + bias_ref[...]

    iota = lax.broadcasted_iota(jnp.int32, (Q_BLK, 128), 1)
    base = m * M_TILE

    bv0 = bv_ref[...]
    bi0 = bi_ref[...]

    m1 = s_ref[:, 0:128]
    a1 = iota + base
    m2 = jnp.full((Q_BLK, 128), NEG, jnp.float32)
    a2 = jnp.zeros((Q_BLK, 128), jnp.int32)
    for j in range(1, NSLICES):
        sj = s_ref[:, j * 128:(j + 1) * 128]
        idxj = iota + (base + j * 128)
        c1 = sj > m1
        c2 = sj > m2
        m2 = jnp.where(c1, m1, jnp.where(c2, sj, m2))
        a2 = jnp.where(c1, a1, jnp.where(c2, idxj, a2))
        m1 = jnp.where(c1, sj, m1)
        a1 = jnp.where(c1, idxj, a1)

    pv = jnp.concatenate([m1, m2, bv0], axis=1)
    pi = jnp.concatenate([a1, a2, bi0], axis=1)
    nv, ni = [], []
    for _ in range(K):
        mx = jnp.max(pv, axis=1, keepdims=True)
        hit = pv >= mx
        cand = jnp.min(jnp.where(hit, pi, IMAX), axis=1, keepdims=True)
        nv.append(mx)
        ni.append(cand)
        pv = jnp.where(hit & (pi == cand), NEG, pv)
    fv = jnp.concatenate(nv, axis=1)
    fi = jnp.concatenate(ni, axis=1)

    tv = fv[:, K - 1:K]
    ti = fi[:, K - 1:K]
    cnt = jnp.zeros((Q_BLK, 1), jnp.int32)
    for j in range(NSLICES):
        sj = s_ref[:, j * 128:(j + 1) * 128]
        idxj = iota + (base + j * 128)
        lg = (sj > tv) | ((sj == tv) & (idxj <= ti))
        cnt += jnp.sum(lg.astype(jnp.int32), axis=1, keepdims=True)
    ft = jnp.zeros((Q_BLK, 1), jnp.int32)
    for j in range(K):
        ft += (fi[:, j:j + 1] >= base).astype(jnp.int32)
    bad = jnp.max(jnp.where(cnt != ft, 1, 0)) > 0
    bv_ref[...] = fv
    bi_ref[...] = fi

    @pl.when(bad)
    def _fallback():
        sl = 1024
        ns = M_TILE // sl
        iota_w = lax.broadcasted_iota(jnp.int32, (Q_BLK, sl), 1)
        cv = bv0
        cin = bi0
        nv2, ni2 = [], []
        for _ in range(K):
            mx = jnp.max(cv, axis=1, keepdims=True)
            for j in range(ns):
                mx = jnp.maximum(
                    mx, jnp.max(s_ref[:, j * sl:(j + 1) * sl], axis=1,
                                keepdims=True))
            cand = jnp.min(jnp.where(cv >= mx, cin, IMAX), axis=1,
                           keepdims=True)
            for j in range(ns):
                sj = s_ref[:, j * sl:(j + 1) * sl]
                idxj = iota_w + (base + j * sl)
                cand = jnp.minimum(
                    cand, jnp.min(jnp.where(sj >= mx, idxj, IMAX), axis=1,
                                  keepdims=True))
            nv2.append(mx)
            ni2.append(cand)
            cv = jnp.where((cv >= mx) & (cin == cand), NEG, cv)
            for j in range(ns):
                sj = s_ref[:, j * sl:(j + 1) * sl]
                idxj = iota_w + (base + j * sl)
                s_ref[:, j * sl:(j + 1) * sl] = jnp.where(
                    (sj >= mx) & (idxj == cand), NEG, sj)
        bv_ref[...] = jnp.concatenate(nv2, axis=1)
        bi_ref[...] = jnp.concatenate(ni2, axis=1)

    @pl.when(m == m_tiles - 1)
    def _():
        vals_ref[...] = bv_ref[...]
        idx_ref[...] = bi_ref[...]


def _run_topk(query, W1, b1, W2, b2, ln_g, ln_b, bank_p, imp_p, bias_p):
    qn, f = query.shape
    m_tiles = bank_p.shape[0] // M_TILE
    body = functools.partial(_topk_body, m_tiles)
    return pl.pallas_call(
        body,
        grid=(qn // Q_BLK, m_tiles),
        in_specs=[
            pl.BlockSpec((Q_BLK, f), lambda q, m: (q, 0)),
            pl.BlockSpec(W1.shape, lambda q, m: (0, 0)),
            pl.BlockSpec(b1.shape, lambda q, m: (0, 0)),
            pl.BlockSpec(W2.shape, lambda q, m: (0, 0)),
            pl.BlockSpec(b2.shape, lambda q, m: (0, 0)),
            pl.BlockSpec(ln_g.shape, lambda q, m: (0, 0)),
            pl.BlockSpec(ln_b.shape, lambda q, m: (0, 0)),
            pl.BlockSpec((M_TILE, bank_p.shape[1]), lambda q, m: (m, 0)),
            pl.BlockSpec((1, M_TILE), lambda q, m: (0, m)),
            pl.BlockSpec((1, M_TILE), lambda q, m: (0, m)),
        ],
        out_specs=[
            pl.BlockSpec((Q_BLK, K), lambda q, m: (q, 0)),
            pl.BlockSpec((Q_BLK, K), lambda q, m: (q, 0)),
        ],
        out_shape=[
            jax.ShapeDtypeStruct((qn, K), jnp.float32),
            jax.ShapeDtypeStruct((qn, K), jnp.int32),
        ],
        scratch_shapes=[
            pltpu.VMEM((Q_BLK, 64), jnp.float32),
            pltpu.VMEM((Q_BLK, M_TILE), jnp.float32),
            pltpu.VMEM((Q_BLK, K), jnp.float32),
            pltpu.VMEM((Q_BLK, K), jnp.int32),
        ],
        compiler_params=pltpu.CompilerParams(
            dimension_semantics=("arbitrary", "arbitrary")),
    )(query, W1, b1, W2, b2, ln_g, ln_b, bank_p, imp_p, bias_p)


def _gather_codes(bank, flat_idx):
    b_total = flat_idx.shape[0]
    d = bank.shape[1]
    nw = 32
    b_per_w = b_total // nw
    mesh = plsc.VectorSubcoreMesh(core_axis_name="c", subcore_axis_name="s")

    @functools.partial(
        pl.kernel, mesh=mesh,
        out_type=jax.ShapeDtypeStruct((b_total, d), jnp.float32),
        compiler_params=pltpu.CompilerParams(use_tc_tiling_on_sc=False),
        scratch_types=[
            pltpu.VMEM((b_per_w,), jnp.int32),
            pltpu.VMEM((b_per_w, d), jnp.float32),
            pltpu.SemaphoreType.DMA,
        ],
    )
    def gk(table_hbm, idx_hbm, out_hbm, idx_v, rows_v, sem):
        wid = lax.axis_index("s") * 2 + lax.axis_index("c")
        base = wid * b_per_w
        pltpu.sync_copy(idx_hbm.at[pl.ds(base, b_per_w)], idx_v)
        pltpu.async_copy(table_hbm.at[idx_v], rows_v, sem).wait()
        pltpu.sync_copy(rows_v, out_hbm.at[pl.ds(base, b_per_w)])

    return gk(bank, flat_idx)


def _decode_body(codes_ref, wd1_ref, bd1_ref, wd2_ref, bd2_ref, out_ref):
    h = jax.nn.gelu(jnp.dot(codes_ref[...], wd1_ref[...],
                            preferred_element_type=jnp.float32) + bd1_ref[...])
    out_ref[...] = jnp.dot(h, wd2_ref[...],
                           preferred_element_type=jnp.float32) + bd2_ref[...]


def _run_decode(codes, Wd1, bd1, Wd2, bd2):
    b_total, d = codes.shape
    f = Wd2.shape[1]
    return pl.pallas_call(
        _decode_body,
        grid=(b_total // DEC_BLK,),
        in_specs=[
            pl.BlockSpec((DEC_BLK, d), lambda i: (i, 0)),
            pl.BlockSpec(Wd1.shape, lambda i: (0, 0)),
            pl.BlockSpec(bd1.shape, lambda i: (0, 0)),
            pl.BlockSpec(Wd2.shape, lambda i: (0, 0)),
            pl.BlockSpec(bd2.shape, lambda i: (0, 0)),
        ],
        out_specs=pl.BlockSpec((DEC_BLK, f), lambda i: (i, 0)),
        out_shape=jax.ShapeDtypeStruct((b_total, f), jnp.float32),
    )(codes, Wd1, bd1, Wd2, bd2)


def kernel(query, W1, b1, W2, b2, ln_g, ln_b, Wd1, bd1, Wd2, bd2,
           memory_bank, memory_importance, top_k):
    qn, f = query.shape
    m_real = memory_bank.shape[0]
    m_tiles = -(-m_real // M_TILE)
    m_pad = m_tiles * M_TILE
    bank_p = jnp.pad(memory_bank, ((0, m_pad - m_real), (0, 0)))
    imp_p = jnp.pad(memory_importance, (0, m_pad - m_real)).reshape(1, m_pad)
    bias_p = jnp.zeros((1, m_pad), jnp.float32).at[:, m_real:].set(NEG)
    vals, idx = _run_topk(query, W1, b1.reshape(1, -1), W2, b2.reshape(1, -1),
                          ln_g.reshape(1, -1), ln_b.reshape(1, -1),
                          bank_p, imp_p, bias_p)
    codes = _gather_codes(memory_bank, idx.reshape(-1))
    decoded = _run_decode(codes, Wd1, bd1.reshape(1, -1), Wd2, bd2.reshape(1, -1))
    return decoded.reshape(qn, K, f), vals, idx


# speedup vs baseline: 1.1245x; 1.1245x over previous
import functools

import jax
import jax.numpy as jnp
from jax import lax
from jax.experimental import pallas as pl
from jax.experimental.pallas import tpu as pltpu
from jax.experimental.pallas import tpu_sc as plsc

K = 16
M_TILE = 4096
NSLICES = M_TILE // 128
Q_BLK = 512
DEC_BLK = 2048
NEG = -3.0e38
IMAX = 2147483647


def _pops(pv, pi):
    nv, ni = [], []
    for _ in range(K):
        mx = jnp.max(pv, axis=1, keepdims=True)
        hit = pv >= mx
        cand = jnp.min(jnp.where(hit, pi, IMAX), axis=1, keepdims=True)
        nv.append(mx)
        ni.append(cand)
        pv = jnp.where(hit & (pi == cand), NEG, pv)
    return jnp.concatenate(nv, axis=1), jnp.concatenate(ni, axis=1)


def _detect(s_ref, iota, base, fv, fi):
    tv = fv[:, K - 1:K]
    ti = fi[:, K - 1:K]
    cnt = jnp.zeros((Q_BLK, 1), jnp.int32)
    for j in range(NSLICES):
        sj = s_ref[:, j * 128:(j + 1) * 128]
        idxj = iota + (base + j * 128)
        lg = (sj > tv) | ((sj == tv) & (idxj <= ti))
        cnt += jnp.sum(lg.astype(jnp.int32), axis=1, keepdims=True)
    ft = jnp.zeros((Q_BLK, 1), jnp.int32)
    for j in range(K):
        ft += (fi[:, j:j + 1] >= base).astype(jnp.int32)
    return jnp.max(jnp.where(cnt != ft, 1, 0)) > 0


def _bucket_top2(s_ref, iota, base, skip_a=None, skip_b=None):
    m1 = jnp.full((Q_BLK, 128), NEG, jnp.float32)
    a1 = jnp.zeros((Q_BLK, 128), jnp.int32)
    m2 = m1
    a2 = a1
    for j in range(NSLICES):
        sj = s_ref[:, j * 128:(j + 1) * 128]
        idxj = iota + (base + j * 128)
        if skip_a is not None:
            sj = jnp.where((idxj == skip_a) | (idxj == skip_b), NEG, sj)
        c1 = sj > m1
        c2 = sj > m2
        m2 = jnp.where(c1, m1, jnp.where(c2, sj, m2))
        a2 = jnp.where(c1, a1, jnp.where(c2, idxj, a2))
        m1 = jnp.where(c1, sj, m1)
        a1 = jnp.where(c1, idxj, a1)
    return m1, a1, m2, a2


def _topk_body(m_tiles, q_ref, w1_ref, b1_ref, w2_ref, b2_ref,
               g_ref, bb_ref, bank_ref, imp_ref, bias_ref, vals_ref, idx_ref,
               zn_ref, s_ref, bv_ref, bi_ref):
    m = pl.program_id(1)

    @pl.when(m == 0)
    def _():
        h = jax.nn.gelu(jnp.dot(q_ref[...], w1_ref[...],
                                preferred_element_type=jnp.float32) + b1_ref[...])
        z = jnp.dot(h, w2_ref[...], preferred_element_type=jnp.float32) + b2_ref[...]
        mu = jnp.mean(z, axis=-1, keepdims=True)
        var = jnp.mean((z - mu) ** 2, axis=-1, keepdims=True)
        z = (z - mu) / jnp.sqrt(var + 1e-5) * g_ref[...] + bb_ref[...]
        zn_ref[...] = z / (jnp.sqrt(jnp.sum(z * z, axis=-1, keepdims=True)) + 1e-8)
        bv_ref[...] = jnp.full((Q_BLK, K), NEG, jnp.float32)
        bi_ref[...] = jnp.zeros((Q_BLK, K), jnp.int32)

    bank = bank_ref[...]
    inv = 1.0 / (jnp.sqrt(jnp.sum(bank * bank, axis=-1, keepdims=True)) + 1e-8)
    mn = bank * inv
    s = lax.dot_general(zn_ref[...], mn, (((1,), (1,)), ((), ())),
                        preferred_element_type=jnp.float32)
    s_ref[...] = s * (0.5 + 0.5 * imp_ref[...]) + bias_ref[...]

    iota = lax.broadcasted_iota(jnp.int32, (Q_BLK, 128), 1)
    base = m * M_TILE

    bv0 = bv_ref[...]
    bi0 = bi_ref[...]

    m1, a1, m2, a2 = _bucket_top2(s_ref, iota, base)
    fv, fi = _pops(jnp.concatenate([m1, m2, bv0], axis=1),
                   jnp.concatenate([a1, a2, bi0], axis=1))
    bad = _detect(s_ref, iota, base, fv, fi)
    bv_ref[...] = fv
    bi_ref[...] = fi

    @pl.when(bad)
    def _repair():
        m3, a3, m4, a4 = _bucket_top2(s_ref, iota, base, a1, a2)
        fv2, fi2 = _pops(jnp.concatenate([m1, m2, m3, m4, bv0], axis=1),
                         jnp.concatenate([a1, a2, a3, a4, bi0], axis=1))
        bad2 = _detect(s_ref, iota, base, fv2, fi2)
        bv_ref[...] = fv2
        bi_ref[...] = fi2

        @pl.when(bad2)
        def _fallback():
            sl = 1024
            ns = M_TILE // sl
            iota_w = lax.broadcasted_iota(jnp.int32, (Q_BLK, sl), 1)
            cv = bv0
            cin = bi0
            nv2, ni2 = [], []
            for _ in range(K):
                mx = jnp.max(cv, axis=1, keepdims=True)
                for j in range(ns):
                    mx = jnp.maximum(
                        mx, jnp.max(s_ref[:, j * sl:(j + 1) * sl], axis=1,
                                    keepdims=True))
                cand = jnp.min(jnp.where(cv >= mx, cin, IMAX), axis=1,
                               keepdims=True)
                for j in range(ns):
                    sj = s_ref[:, j * sl:(j + 1) * sl]
                    idxj = iota_w + (base + j * sl)
                    cand = jnp.minimum(
                        cand, jnp.min(jnp.where(sj >= mx, idxj, IMAX),
                                      axis=1, keepdims=True))
                nv2.append(mx)
                ni2.append(cand)
                cv = jnp.where((cv >= mx) & (cin == cand), NEG, cv)
                for j in range(ns):
                    sj = s_ref[:, j * sl:(j + 1) * sl]
                    idxj = iota_w + (base + j * sl)
                    s_ref[:, j * sl:(j + 1) * sl] = jnp.where(
                        (sj >= mx) & (idxj == cand), NEG, sj)
            bv_ref[...] = jnp.concatenate(nv2, axis=1)
            bi_ref[...] = jnp.concatenate(ni2, axis=1)

    @pl.when(m == m_tiles - 1)
    def _():
        vals_ref[...] = bv_ref[...]
        idx_ref[...] = bi_ref[...]


def _run_topk(query, W1, b1, W2, b2, ln_g, ln_b, bank_p, imp_p, bias_p):
    qn, f = query.shape
    m_tiles = bank_p.shape[0] // M_TILE
    body = functools.partial(_topk_body, m_tiles)
    return pl.pallas_call(
        body,
        grid=(qn // Q_BLK, m_tiles),
        in_specs=[
            pl.BlockSpec((Q_BLK, f), lambda q, m: (q, 0)),
            pl.BlockSpec(W1.shape, lambda q, m: (0, 0)),
            pl.BlockSpec(b1.shape, lambda q, m: (0, 0)),
            pl.BlockSpec(W2.shape, lambda q, m: (0, 0)),
            pl.BlockSpec(b2.shape, lambda q, m: (0, 0)),
            pl.BlockSpec(ln_g.shape, lambda q, m: (0, 0)),
            pl.BlockSpec(ln_b.shape, lambda q, m: (0, 0)),
            pl.BlockSpec((M_TILE, bank_p.shape[1]), lambda q, m: (m, 0)),
            pl.BlockSpec((1, M_TILE), lambda q, m: (0, m)),
            pl.BlockSpec((1, M_TILE), lambda q, m: (0, m)),
        ],
        out_specs=[
            pl.BlockSpec((Q_BLK, K), lambda q, m: (q, 0)),
            pl.BlockSpec((Q_BLK, K), lambda q, m: (q, 0)),
        ],
        out_shape=[
            jax.ShapeDtypeStruct((qn, K), jnp.float32),
            jax.ShapeDtypeStruct((qn, K), jnp.int32),
        ],
        scratch_shapes=[
            pltpu.VMEM((Q_BLK, 64), jnp.float32),
            pltpu.VMEM((Q_BLK, M_TILE), jnp.float32),
            pltpu.VMEM((Q_BLK, K), jnp.float32),
            pltpu.VMEM((Q_BLK, K), jnp.int32),
        ],
        compiler_params=pltpu.CompilerParams(
            dimension_semantics=("arbitrary", "arbitrary")),
    )(query, W1, b1, W2, b2, ln_g, ln_b, bank_p, imp_p, bias_p)


def _gather_codes(bank, flat_idx):
    b_total = flat_idx.shape[0]
    d = bank.shape[1]
    nw = 32
    b_per_w = b_total // nw
    mesh = plsc.VectorSubcoreMesh(core_axis_name="c", subcore_axis_name="s")

    @functools.partial(
        pl.kernel, mesh=mesh,
        out_type=jax.ShapeDtypeStruct((b_total, d), jnp.float32),
        compiler_params=pltpu.CompilerParams(use_tc_tiling_on_sc=False),
        scratch_types=[
            pltpu.VMEM((b_per_w,), jnp.int32),
            pltpu.VMEM((b_per_w, d), jnp.float32),
            pltpu.SemaphoreType.DMA,
        ],
    )
    def gk(table_hbm, idx_hbm, out_hbm, idx_v, rows_v, sem):
        wid = lax.axis_index("s") * 2 + lax.axis_index("c")
        base = wid * b_per_w
        pltpu.sync_copy(idx_hbm.at[pl.ds(base, b_per_w)], idx_v)
        pltpu.async_copy(table_hbm.at[idx_v], rows_v, sem).wait()
        pltpu.sync_copy(rows_v, out_hbm.at[pl.ds(base, b_per_w)])

    return gk(bank, flat_idx)


def _decode_body(codes_ref, wd1_ref, bd1_ref, wd2_ref, bd2_ref, out_ref):
    h = jax.nn.gelu(jnp.dot(codes_ref[...], wd1_ref[...],
                            preferred_element_type=jnp.float32) + bd1_ref[...])
    out_ref[...] = jnp.dot(h, wd2_ref[...],
                           preferred_element_type=jnp.float32) + bd2_ref[...]


def _run_decode(codes, Wd1, bd1, Wd2, bd2):
    b_total, d = codes.shape
    f = Wd2.shape[1]
    return pl.pallas_call(
        _decode_body,
        grid=(b_total // DEC_BLK,),
        in_specs=[
            pl.BlockSpec((DEC_BLK, d), lambda i: (i, 0)),
            pl.BlockSpec(Wd1.shape, lambda i: (0, 0)),
            pl.BlockSpec(bd1.shape, lambda i: (0, 0)),
            pl.BlockSpec(Wd2.shape, lambda i: (0, 0)),
            pl.BlockSpec(bd2.shape, lambda i: (0, 0)),
        ],
        out_specs=pl.BlockSpec((DEC_BLK, f), lambda i: (i, 0)),
        out_shape=jax.ShapeDtypeStruct((b_total, f), jnp.float32),
    )(codes, Wd1, bd1, Wd2, bd2)


def kernel(query, W1, b1, W2, b2, ln_g, ln_b, Wd1, bd1, Wd2, bd2,
           memory_bank, memory_importance, top_k):
    qn, f = query.shape
    m_real = memory_bank.shape[0]
    m_tiles = -(-m_real // M_TILE)
    m_pad = m_tiles * M_TILE
    bank_p = jnp.pad(memory_bank, ((0, m_pad - m_real), (0, 0)))
    imp_p = jnp.pad(memory_importance, (0, m_pad - m_real)).reshape(1, m_pad)
    bias_p = jnp.zeros((1, m_pad), jnp.float32).at[:, m_real:].set(NEG)
    vals, idx = _run_topk(query, W1, b1.reshape(1, -1), W2, b2.reshape(1, -1),
                          ln_g.reshape(1, -1), ln_b.reshape(1, -1),
                          bank_p, imp_p, bias_p)
    codes = _gather_codes(memory_bank, idx.reshape(-1))
    decoded = _run_decode(codes, Wd1, bd1.reshape(1, -1), Wd2, bd2.reshape(1, -1))
    return decoded.reshape(qn, K, f), vals, idx


# speedup vs baseline: 1.7387x; 1.5462x over previous
import functools

import jax
import jax.numpy as jnp
from jax import lax
from jax.experimental import pallas as pl
from jax.experimental.pallas import tpu as pltpu
from jax.experimental.pallas import tpu_sc as plsc

K = 16
M_TILE = 4096
NSLICES = M_TILE // 128
QA = 1024
QC = 512
DEC_BLK = 2048
NEG = -3.0e38
IMAX = 2147483647


def _encode(q_ref, w1_ref, b1_ref, w2_ref, b2_ref, g_ref, bb_ref):
    h = jax.nn.gelu(jnp.dot(q_ref[...], w1_ref[...],
                            preferred_element_type=jnp.float32) + b1_ref[...])
    z = jnp.dot(h, w2_ref[...], preferred_element_type=jnp.float32) + b2_ref[...]
    mu = jnp.mean(z, axis=-1, keepdims=True)
    var = jnp.mean((z - mu) ** 2, axis=-1, keepdims=True)
    z = (z - mu) / jnp.sqrt(var + 1e-5) * g_ref[...] + bb_ref[...]
    return z / (jnp.sqrt(jnp.sum(z * z, axis=-1, keepdims=True)) + 1e-8)


def _sim_chunk(zn, bank_ref, imp_ref, bias_ref, c):
    bank = bank_ref[c * 1024:(c + 1) * 1024, :]
    inv = 1.0 / (jnp.sqrt(jnp.sum(bank * bank, axis=-1, keepdims=True)) + 1e-8)
    sc = lax.dot_general(zn, bank * inv, (((1,), (1,)), ((), ())),
                         preferred_element_type=jnp.float32)
    w = 0.5 + 0.5 * imp_ref[:, c * 1024:(c + 1) * 1024]
    return sc * w + bias_ref[:, c * 1024:(c + 1) * 1024]


def _pops(pv_ref, pi_ref, width, q_blk):
    nv, ni = [], []
    for _ in range(K):
        pv = pv_ref[:, :width]
        pi = pi_ref[:, :width]
        mx = jnp.max(pv, axis=1, keepdims=True)
        hit = pv >= mx
        cand = jnp.min(jnp.where(hit, pi, IMAX), axis=1, keepdims=True)
        nv.append(mx)
        ni.append(cand)
        pv_ref[:, :width] = jnp.where(hit & (pi == cand), NEG, pv)
    return jnp.concatenate(nv, axis=1), jnp.concatenate(ni, axis=1)


def _bucket_top2(s_ref, iota, base, q_blk, skip_a=None, skip_b=None):
    m1 = jnp.full((q_blk, 128), NEG, jnp.float32)
    a1 = jnp.zeros((q_blk, 128), jnp.int32)
    m2 = m1
    a2 = a1
    for j in range(NSLICES):
        sj = s_ref[:, j * 128:(j + 1) * 128]
        idxj = iota + (base + j * 128)
        if skip_a is not None:
            sj = jnp.where((idxj == skip_a) | (idxj == skip_b), NEG, sj)
        c1 = sj > m1
        c2 = sj > m2
        m2 = jnp.where(c1, m1, jnp.where(c2, sj, m2))
        a2 = jnp.where(c1, a1, jnp.where(c2, idxj, a2))
        m1 = jnp.where(c1, sj, m1)
        a1 = jnp.where(c1, idxj, a1)
    return m1, a1, m2, a2


def _detect(s_ref, iota, base, fv, fi, q_blk):
    tv = fv[:, K - 1:K]
    ti = fi[:, K - 1:K]
    cnt = jnp.zeros((q_blk, 1), jnp.int32)
    for j in range(NSLICES):
        sj = s_ref[:, j * 128:(j + 1) * 128]
        idxj = iota + (base + j * 128)
        lg = (sj > tv) | ((sj == tv) & (idxj <= ti))
        cnt += jnp.sum(lg.astype(jnp.int32), axis=1, keepdims=True)
    ft = jnp.zeros((q_blk, 1), jnp.int32)
    for j in range(K):
        ft += (fi[:, j:j + 1] >= base).astype(jnp.int32)
    return jnp.max(jnp.where(cnt != ft, 1, 0)) > 0


def _fast_body(m_tiles, q_ref, w1_ref, b1_ref, w2_ref, b2_ref, g_ref, bb_ref,
               bank_ref, imp_ref, bias_ref, vals_ref, idx_ref, zn_out_ref,
               s_ref, bv_ref, bi_ref, pv_ref, pi_ref):
    m = pl.program_id(0)

    @pl.when(m == 0)
    def _():
        zn_out_ref[...] = _encode(q_ref, w1_ref, b1_ref, w2_ref, b2_ref,
                                  g_ref, bb_ref)
        bv_ref[...] = jnp.full((QA, K), NEG, jnp.float32)
        bi_ref[...] = jnp.zeros((QA, K), jnp.int32)

    zn = zn_out_ref[...]
    for c in range(M_TILE // 1024):
        s_ref[:, c * 1024:(c + 1) * 1024] = _sim_chunk(
            zn, bank_ref, imp_ref, bias_ref, c)

    iota = lax.broadcasted_iota(jnp.int32, (QA, 128), 1)
    base = m * M_TILE
    m1, a1, m2, a2 = _bucket_top2(s_ref, iota, base, QA)
    pv_ref[:, 0:128] = m1
    pv_ref[:, 128:256] = m2
    pv_ref[:, 256:272] = bv_ref[...]
    pi_ref[:, 0:128] = a1
    pi_ref[:, 128:256] = a2
    pi_ref[:, 256:272] = bi_ref[...]
    fv, fi = _pops(pv_ref, pi_ref, 272, QA)
    bv_ref[...] = fv
    bi_ref[...] = fi

    @pl.when(m == m_tiles - 1)
    def _():
        vals_ref[...] = bv_ref[...]
        idx_ref[...] = bi_ref[...]


def _run_fast(query, W1, b1, W2, b2, ln_g, ln_b, bank_p, imp_p, bias_p):
    qn, f = query.shape
    m_tiles = bank_p.shape[0] // M_TILE
    body = functools.partial(_fast_body, m_tiles)
    return pl.pallas_call(
        body,
        grid=(m_tiles,),
        in_specs=[
            pl.BlockSpec((QA, f), lambda m: (0, 0)),
            pl.BlockSpec(W1.shape, lambda m: (0, 0)),
            pl.BlockSpec(b1.shape, lambda m: (0, 0)),
            pl.BlockSpec(W2.shape, lambda m: (0, 0)),
            pl.BlockSpec(b2.shape, lambda m: (0, 0)),
            pl.BlockSpec(ln_g.shape, lambda m: (0, 0)),
            pl.BlockSpec(ln_b.shape, lambda m: (0, 0)),
            pl.BlockSpec((M_TILE, bank_p.shape[1]), lambda m: (m, 0)),
            pl.BlockSpec((1, M_TILE), lambda m: (0, m)),
            pl.BlockSpec((1, M_TILE), lambda m: (0, m)),
        ],
        out_specs=[
            pl.BlockSpec((QA, K), lambda m: (0, 0)),
            pl.BlockSpec((QA, K), lambda m: (0, 0)),
            pl.BlockSpec((QA, 64), lambda m: (0, 0)),
        ],
        out_shape=[
            jax.ShapeDtypeStruct((qn, K), jnp.float32),
            jax.ShapeDtypeStruct((qn, K), jnp.int32),
            jax.ShapeDtypeStruct((qn, 64), jnp.float32),
        ],
        scratch_shapes=[
            pltpu.VMEM((QA, M_TILE), jnp.float32),
            pltpu.VMEM((QA, K), jnp.float32),
            pltpu.VMEM((QA, K), jnp.int32),
            pltpu.VMEM((QA, 272), jnp.float32),
            pltpu.VMEM((QA, 272), jnp.int32),
        ],
        compiler_params=pltpu.CompilerParams(
            dimension_semantics=("arbitrary",)),
    )(query, W1, b1, W2, b2, ln_g, ln_b, bank_p, imp_p, bias_p)


def _verify_body(m_tiles, zn_ref, vals_ref, idx_ref, bank_ref, imp_ref,
                 bias_ref, cnt_ref, acc_ref):
    m = pl.program_id(0)

    @pl.when(m == 0)
    def _():
        acc_ref[...] = jnp.zeros((QA, 1), jnp.int32)

    zn = zn_ref[...]
    tv = vals_ref[:, K - 1:K]
    ti = idx_ref[:, K - 1:K]
    iota = lax.broadcasted_iota(jnp.int32, (QA, 1024), 1)
    base = m * M_TILE
    acc = acc_ref[...]
    for c in range(M_TILE // 1024):
        sc = _sim_chunk(zn, bank_ref, imp_ref, bias_ref, c)
        idxc = iota + (base + c * 1024)
        lg = (sc > tv) | ((sc == tv) & (idxc <= ti))
        acc = acc + jnp.sum(lg.astype(jnp.int32), axis=1, keepdims=True)
    acc_ref[...] = acc

    @pl.when(m == m_tiles - 1)
    def _():
        cnt_ref[...] = acc_ref[...]


def _run_verify(zn, vals, idx, bank_p, imp_p, bias_p):
    qn = zn.shape[0]
    m_tiles = bank_p.shape[0] // M_TILE
    body = functools.partial(_verify_body, m_tiles)
    return pl.pallas_call(
        body,
        grid=(m_tiles,),
        in_specs=[
            pl.BlockSpec((QA, 64), lambda m: (0, 0)),
            pl.BlockSpec((QA, K), lambda m: (0, 0)),
            pl.BlockSpec((QA, K), lambda m: (0, 0)),
            pl.BlockSpec((M_TILE, bank_p.shape[1]), lambda m: (m, 0)),
            pl.BlockSpec((1, M_TILE), lambda m: (0, m)),
            pl.BlockSpec((1, M_TILE), lambda m: (0, m)),
        ],
        out_specs=pl.BlockSpec((QA, 1), lambda m: (0, 0)),
        out_shape=jax.ShapeDtypeStruct((qn, 1), jnp.int32),
        scratch_shapes=[pltpu.VMEM((QA, 1), jnp.int32)],
        compiler_params=pltpu.CompilerParams(
            dimension_semantics=("arbitrary",)),
    )(zn, vals, idx, bank_p, imp_p, bias_p)


def _exact_body(m_tiles, q_ref, w1_ref, b1_ref, w2_ref, b2_ref, g_ref, bb_ref,
                bank_ref, imp_ref, bias_ref, vals_ref, idx_ref,
                zn_ref, s_ref, bv_ref, bi_ref, pv_ref, pi_ref):
    m = pl.program_id(1)

    @pl.when(m == 0)
    def _():
        zn_ref[...] = _encode(q_ref, w1_ref, b1_ref, w2_ref, b2_ref,
                              g_ref, bb_ref)
        bv_ref[...] = jnp.full((QC, K), NEG, jnp.float32)
        bi_ref[...] = jnp.zeros((QC, K), jnp.int32)

    zn = zn_ref[...]
    for c in range(M_TILE // 1024):
        s_ref[:, c * 1024:(c + 1) * 1024] = _sim_chunk(
            zn, bank_ref, imp_ref, bias_ref, c)

    iota = lax.broadcasted_iota(jnp.int32, (QC, 128), 1)
    base = m * M_TILE
    bv0 = bv_ref[...]
    bi0 = bi_ref[...]

    m1, a1, m2, a2 = _bucket_top2(s_ref, iota, base, QC)
    pv_ref[:, 0:128] = m1
    pv_ref[:, 128:256] = m2
    pv_ref[:, 256:272] = bv0
    pi_ref[:, 0:128] = a1
    pi_ref[:, 128:256] = a2
    pi_ref[:, 256:272] = bi0
    fv, fi = _pops(pv_ref, pi_ref, 272, QC)
    bad = _detect(s_ref, iota, base, fv, fi, QC)
    bv_ref[...] = fv
    bi_ref[...] = fi

    @pl.when(bad)
    def _repair():
        r1, b1_, r2, b2_ = _bucket_top2(s_ref, iota, base, QC)
        m3, a3, m4, a4 = _bucket_top2(s_ref, iota, base, QC, b1_, b2_)
        pv_ref[:, 0:128] = r1
        pv_ref[:, 128:256] = r2
        pv_ref[:, 256:384] = m3
        pv_ref[:, 384:512] = m4
        pv_ref[:, 512:528] = bv0
        pi_ref[:, 0:128] = b1_
        pi_ref[:, 128:256] = b2_
        pi_ref[:, 256:384] = a3
        pi_ref[:, 384:512] = a4
        pi_ref[:, 512:528] = bi0
        fv2, fi2 = _pops(pv_ref, pi_ref, 528, QC)
        bad2 = _detect(s_ref, iota, base, fv2, fi2, QC)
        bv_ref[...] = fv2
        bi_ref[...] = fi2

        @pl.when(bad2)
        def _fallback():
            sl = 1024
            ns = M_TILE // sl
            iota_w = lax.broadcasted_iota(jnp.int32, (QC, sl), 1)
            cv = bv0
            cin = bi0
            nv2, ni2 = [], []
            for _ in range(K):
                mx = jnp.max(cv, axis=1, keepdims=True)
                for j in range(ns):
                    mx = jnp.maximum(
                        mx, jnp.max(s_ref[:, j * sl:(j + 1) * sl], axis=1,
                                    keepdims=True))
                cand = jnp.min(jnp.where(cv >= mx, cin, IMAX), axis=1,
                               keepdims=True)
                for j in range(ns):
                    sj = s_ref[:, j * sl:(j + 1) * sl]
                    idxj = iota_w + (base + j * sl)
                    cand = jnp.minimum(
                        cand, jnp.min(jnp.where(sj >= mx, idxj, IMAX),
                                      axis=1, keepdims=True))
                nv2.append(mx)
                ni2.append(cand)
                cv = jnp.where((cv >= mx) & (cin == cand), NEG, cv)
                for j in range(ns):
                    sj = s_ref[:, j * sl:(j + 1) * sl]
                    idxj = iota_w + (base + j * sl)
                    s_ref[:, j * sl:(j + 1) * sl] = jnp.where(
                        (sj >= mx) & (idxj == cand), NEG, sj)
            bv_ref[...] = jnp.concatenate(nv2, axis=1)
            bi_ref[...] = jnp.concatenate(ni2, axis=1)

    @pl.when(m == m_tiles - 1)
    def _():
        vals_ref[...] = bv_ref[...]
        idx_ref[...] = bi_ref[...]


def _run_exact(query, W1, b1, W2, b2, ln_g, ln_b, bank_p, imp_p, bias_p):
    qn, f = query.shape
    m_tiles = bank_p.shape[0] // M_TILE
    body = functools.partial(_exact_body, m_tiles)
    return pl.pallas_call(
        body,
        grid=(qn // QC, m_tiles),
        in_specs=[
            pl.BlockSpec((QC, f), lambda q, m: (q, 0)),
            pl.BlockSpec(W1.shape, lambda q, m: (0, 0)),
            pl.BlockSpec(b1.shape, lambda q, m: (0, 0)),
            pl.BlockSpec(W2.shape, lambda q, m: (0, 0)),
            pl.BlockSpec(b2.shape, lambda q, m: (0, 0)),
            pl.BlockSpec(ln_g.shape, lambda q, m: (0, 0)),
            pl.BlockSpec(ln_b.shape, lambda q, m: (0, 0)),
            pl.BlockSpec((M_TILE, bank_p.shape[1]), lambda q, m: (m, 0)),
            pl.BlockSpec((1, M_TILE), lambda q, m: (0, m)),
            pl.BlockSpec((1, M_TILE), lambda q, m: (0, m)),
        ],
        out_specs=[
            pl.BlockSpec((QC, K), lambda q, m: (q, 0)),
            pl.BlockSpec((QC, K), lambda q, m: (q, 0)),
        ],
        out_shape=[
            jax.ShapeDtypeStruct((qn, K), jnp.float32),
            jax.ShapeDtypeStruct((qn, K), jnp.int32),
        ],
        scratch_shapes=[
            pltpu.VMEM((QC, 64), jnp.float32),
            pltpu.VMEM((QC, M_TILE), jnp.float32),
            pltpu.VMEM((QC, K), jnp.float32),
            pltpu.VMEM((QC, K), jnp.int32),
            pltpu.VMEM((QC, 528), jnp.float32),
            pltpu.VMEM((QC, 528), jnp.int32),
        ],
        compiler_params=pltpu.CompilerParams(
            dimension_semantics=("arbitrary", "arbitrary")),
    )(query, W1, b1, W2, b2, ln_g, ln_b, bank_p, imp_p, bias_p)


def _gather_codes(bank, flat_idx):
    b_total = flat_idx.shape[0]
    d = bank.shape[1]
    nw = 32
    b_per_w = b_total // nw
    mesh = plsc.VectorSubcoreMesh(core_axis_name="c", subcore_axis_name="s")

    @functools.partial(
        pl.kernel, mesh=mesh,
        out_type=jax.ShapeDtypeStruct((b_total, d), jnp.float32),
        compiler_params=pltpu.CompilerParams(use_tc_tiling_on_sc=False),
        scratch_types=[
            pltpu.VMEM((b_per_w,), jnp.int32),
            pltpu.VMEM((b_per_w, d), jnp.float32),
            pltpu.SemaphoreType.DMA,
        ],
    )
    def gk(table_hbm, idx_hbm, out_hbm, idx_v, rows_v, sem):
        wid = lax.axis_index("s") * 2 + lax.axis_index("c")
        base = wid * b_per_w
        pltpu.sync_copy(idx_hbm.at[pl.ds(base, b_per_w)], idx_v)
        pltpu.async_copy(table_hbm.at[idx_v], rows_v, sem).wait()
        pltpu.sync_copy(rows_v, out_hbm.at[pl.ds(base, b_per_w)])

    return gk(bank, flat_idx)


def _decode_body(codes_ref, wd1_ref, bd1_ref, wd2_ref, bd2_ref, out_ref):
    h = jax.nn.gelu(jnp.dot(codes_ref[...], wd1_ref[...],
                            preferred_element_type=jnp.float32) + bd1_ref[...])
    out_ref[...] = jnp.dot(h, wd2_ref[...],
                           preferred_element_type=jnp.float32) + bd2_ref[...]


def _run_decode(codes, Wd1, bd1, Wd2, bd2):
    b_total, d = codes.shape
    f = Wd2.shape[1]
    return pl.pallas_call(
        _decode_body,
        grid=(b_total // DEC_BLK,),
        in_specs=[
            pl.BlockSpec((DEC_BLK, d), lambda i: (i, 0)),
            pl.BlockSpec(Wd1.shape, lambda i: (0, 0)),
            pl.BlockSpec(bd1.shape, lambda i: (0, 0)),
            pl.BlockSpec(Wd2.shape, lambda i: (0, 0)),
            pl.BlockSpec(bd2.shape, lambda i: (0, 0)),
        ],
        out_specs=pl.BlockSpec((DEC_BLK, f), lambda i: (i, 0)),
        out_shape=jax.ShapeDtypeStruct((b_total, f), jnp.float32),
    )(codes, Wd1, bd1, Wd2, bd2)


def kernel(query, W1, b1, W2, b2, ln_g, ln_b, Wd1, bd1, Wd2, bd2,
           memory_bank, memory_importance, top_k):
    qn, f = query.shape
    m_real = memory_bank.shape[0]
    m_tiles = -(-m_real // M_TILE)
    m_pad = m_tiles * M_TILE
    bank_p = jnp.pad(memory_bank, ((0, m_pad - m_real), (0, 0)))
    imp_p = jnp.pad(memory_importance, (0, m_pad - m_real)).reshape(1, m_pad)
    bias_p = jnp.zeros((1, m_pad), jnp.float32).at[:, m_real:].set(NEG)
    args = (query, W1, b1.reshape(1, -1), W2, b2.reshape(1, -1),
            ln_g.reshape(1, -1), ln_b.reshape(1, -1), bank_p, imp_p, bias_p)
    vals_f, idx_f, zn = _run_fast(*args)
    cnt = _run_verify(zn, vals_f, idx_f, bank_p, imp_p, bias_p)
    bad = jnp.max(jnp.abs(cnt - K)) > 0
    vals, idx = lax.cond(bad,
                         lambda: _run_exact(*args),
                         lambda: (vals_f, idx_f))
    codes = _gather_codes(memory_bank, idx.reshape(-1))
    decoded = _run_decode(codes, Wd1, bd1.reshape(1, -1), Wd2, bd2.reshape(1, -1))
    return decoded.reshape(qn, K, f), vals, idx
